# baseline (speedup 1.0000x reference)
import functools

import jax
import jax.numpy as jnp
from jax import lax
from jax.experimental import pallas as pl
from jax.experimental.pallas import tpu as pltpu, tpu_sc as plsc

_NUM_STYLES = 1000
_DIM = 64
_BATCH = 16384

_NC = 2
_NS = 16
_NW = _NC * _NS
_BPW = _BATCH // _NW
_HPW = _BPW // 2
_CHUNK = 128


def _emb_body(idx_hbm, table_hbm, out_hbm, table_s, idx_v, idx_e, idx_o,
              rows_e, rows_o, gsem, osem):
    cid = lax.axis_index("c")
    sid = lax.axis_index("s")
    wid = sid * _NC + cid
    base = wid * _BPW

    @pl.when(sid == 0)
    def _stage_table():
        pltpu.sync_copy(table_hbm, table_s)

    pltpu.sync_copy(idx_hbm.at[pl.ds(base, _BPW)], idx_v)

    for i in range(_HPW // 16):
        pos = lax.iota(jnp.int32, 16) * 2 + i * 32
        idx_e[pl.ds(i * 16, 16)] = plsc.load_gather(idx_v, [pos])
        idx_o[pl.ds(i * 16, 16)] = plsc.load_gather(idx_v, [pos + 1])

    plsc.subcore_barrier()

    gathers = []
    for c in range(_HPW // _CHUNK):
        gathers.append(
            pltpu.async_copy(
                table_s.at[idx_e.at[pl.ds(c * _CHUNK, _CHUNK)]],
                rows_e.at[pl.ds(c * _CHUNK, _CHUNK)],
                gsem,
            )
        )
        gathers.append(
            pltpu.async_copy(
                table_s.at[idx_o.at[pl.ds(c * _CHUNK, _CHUNK)]],
                rows_o.at[pl.ds(c * _CHUNK, _CHUNK)],
                gsem,
            )
        )
    for g in gathers:
        g.wait()
    t0 = wid * _HPW
    ce = pltpu.async_copy(
        rows_e, out_hbm.at[pl.ds(t0, _HPW), pl.ds(0, _DIM)], osem)
    co = pltpu.async_copy(
        rows_o, out_hbm.at[pl.ds(t0, _HPW), pl.ds(_DIM, _DIM)], osem)
    ce.wait()
    co.wait()


_emb = functools.partial(
    pl.kernel,
    out_type=jax.ShapeDtypeStruct((_BATCH // 2, 2 * _DIM), jnp.float32),
    mesh=plsc.VectorSubcoreMesh(core_axis_name="c", subcore_axis_name="s"),
    scratch_types=[
        pltpu.VMEM_SHARED((_NUM_STYLES, _DIM), jnp.float32),
        pltpu.VMEM((_BPW,), jnp.int32),
        pltpu.VMEM((_HPW,), jnp.int32),
        pltpu.VMEM((_HPW,), jnp.int32),
        pltpu.VMEM((_HPW, _DIM), jnp.float32),
        pltpu.VMEM((_HPW, _DIM), jnp.float32),
        pltpu.SemaphoreType.DMA,
        pltpu.SemaphoreType.DMA,
    ],
    compiler_params=pltpu.CompilerParams(
        use_tc_tiling_on_sc=False, needs_layout_passes=False
    ),
)(_emb_body)


def _relayout_body(x_ref, o_ref):
    x = x_ref[...]
    a = jnp.repeat(x[:, :_DIM], 2, axis=0)
    b = jnp.repeat(x[:, _DIM:], 2, axis=0)
    n = o_ref.shape[0]
    even_row = jax.lax.broadcasted_iota(jnp.int32, (n, _DIM), 0) % 2 == 0
    o_ref[...] = jnp.where(even_row, a, b)


_relayout = pl.pallas_call(
    _relayout_body,
    out_shape=jax.ShapeDtypeStruct((_BATCH, _DIM), jnp.float32),
    grid=(8,),
    in_specs=[pl.BlockSpec((_BATCH // 16, 2 * _DIM), lambda g: (g, 0))],
    out_specs=pl.BlockSpec((_BATCH // 8, _DIM), lambda g: (g, 0)),
)


def kernel(style_id, embed_weight):
    out2 = _emb(style_id.astype(jnp.int32), embed_weight)
    return _relayout(out2)


# speedup vs baseline: 1.0584x; 1.0584x over previous
import functools

import jax
import jax.numpy as jnp
from jax import lax
from jax.experimental import pallas as pl
from jax.experimental.pallas import tpu as pltpu, tpu_sc as plsc

_NUM_STYLES = 1000
_DIM = 64
_BATCH = 16384

_NC = 2
_NS = 16
_NW = _NC * _NS
_BPW = _BATCH // _NW
_CHUNK = 128
_NCHUNK = _BPW // _CHUNK


def _emb_body(idx_hbm, table_hbm, out_hbm, table_s, idx_v, rows_v, gsem, osem):
    cid = lax.axis_index("c")
    sid = lax.axis_index("s")
    base = (sid * _NC + cid) * _BPW

    @pl.when(sid == 0)
    def _stage_table():
        pltpu.sync_copy(table_hbm, table_s)

    pltpu.sync_copy(idx_hbm.at[pl.ds(base, _BPW)], idx_v)
    plsc.subcore_barrier()

    gathers = []
    for j in range(_NCHUNK):
        gathers.append(
            pltpu.async_copy(
                table_s.at[idx_v.at[pl.ds(j * _CHUNK, _CHUNK)]],
                rows_v.at[pl.ds(j * _CHUNK, _CHUNK)],
                gsem,
            )
        )
    outs = []
    for j in range(_NCHUNK):
        gathers[j].wait()
        outs.append(
            pltpu.async_copy(
                rows_v.at[pl.ds(j * _CHUNK, _CHUNK)],
                out_hbm.at[pl.ds(base + j * _CHUNK, _CHUNK), pl.ds(0, _DIM)],
                osem,
            )
        )
    for c in outs:
        c.wait()


_emb = functools.partial(
    pl.kernel,
    out_type=jax.ShapeDtypeStruct((_BATCH, 2 * _DIM), jnp.float32),
    mesh=plsc.VectorSubcoreMesh(core_axis_name="c", subcore_axis_name="s"),
    scratch_types=[
        pltpu.VMEM_SHARED((_NUM_STYLES, _DIM), jnp.float32),
        pltpu.VMEM((_BPW,), jnp.int32),
        pltpu.VMEM((_BPW, _DIM), jnp.float32),
        pltpu.SemaphoreType.DMA,
        pltpu.SemaphoreType.DMA,
    ],
    compiler_params=pltpu.CompilerParams(
        use_tc_tiling_on_sc=False, needs_layout_passes=False
    ),
)(_emb_body)


def _take_left_body(x_ref, o_ref):
    o_ref[...] = x_ref[:, :_DIM]


_take_left = pl.pallas_call(
    _take_left_body,
    out_shape=jax.ShapeDtypeStruct((_BATCH, _DIM), jnp.float32),
    grid=(8,),
    in_specs=[pl.BlockSpec((_BATCH // 8, 2 * _DIM), lambda g: (g, 0))],
    out_specs=pl.BlockSpec((_BATCH // 8, _DIM), lambda g: (g, 0)),
)


def kernel(style_id, embed_weight):
    padded = _emb(style_id.astype(jnp.int32), embed_weight)
    return _take_left(padded)


# speedup vs baseline: 1.3774x; 1.3014x over previous
import functools

import jax
import jax.numpy as jnp
from jax import lax
from jax.experimental import pallas as pl
from jax.experimental.pallas import tpu as pltpu, tpu_sc as plsc

_NUM_STYLES = 1000
_DIM = 64
_BATCH = 16384

_NC = 2
_NS = 16
_NW = _NC * _NS
_BPW = _BATCH // _NW
_CHUNK = 128
_NCHUNK = _BPW // _CHUNK


def _emb_body(idx_hbm, table_hbm, out_hbm, table_s, idx_v, rows_v, gsem, osem):
    cid = lax.axis_index("c")
    sid = lax.axis_index("s")
    base = (sid * _NC + cid) * _BPW

    @pl.when(sid == 0)
    def _stage_table():
        pltpu.sync_copy(table_hbm, table_s)

    pltpu.sync_copy(idx_hbm.at[pl.ds(base, _BPW)], idx_v)
    plsc.subcore_barrier()

    gathers = []
    for j in range(_NCHUNK):
        gathers.append(
            pltpu.async_copy(
                table_s.at[idx_v.at[pl.ds(j * _CHUNK, _CHUNK)]],
                rows_v.at[pl.ds(j * _CHUNK, _CHUNK)],
                gsem,
            )
        )
    outs = []
    for j in range(_NCHUNK):
        gathers[j].wait()
        outs.append(
            pltpu.async_copy(
                rows_v.at[pl.ds(j * _CHUNK, _CHUNK)],
                out_hbm.at[pl.ds(base + j * _CHUNK, _CHUNK), pl.ds(0, _DIM)],
                osem,
            )
        )
    for c in outs:
        c.wait()


_emb = functools.partial(
    pl.kernel,
    out_type=jax.ShapeDtypeStruct((_BATCH, 2 * _DIM), jnp.float32),
    mesh=plsc.VectorSubcoreMesh(core_axis_name="c", subcore_axis_name="s"),
    scratch_types=[
        pltpu.VMEM_SHARED((_NUM_STYLES, _DIM), jnp.float32),
        pltpu.VMEM((_BPW,), jnp.int32),
        pltpu.VMEM((_BPW, _DIM), jnp.float32),
        pltpu.SemaphoreType.DMA,
        pltpu.SemaphoreType.DMA,
    ],
    compiler_params=pltpu.CompilerParams(
        use_tc_tiling_on_sc=False, needs_layout_passes=False
    ),
)(_emb_body)


def _take_left_body(x_ref, o_ref):
    o_ref[...] = x_ref[:, :_DIM]


_take_left = pl.pallas_call(
    _take_left_body,
    out_shape=jax.ShapeDtypeStruct((_BATCH, _DIM), jnp.float32),
    grid=(8,),
    in_specs=[pl.BlockSpec((_BATCH // 8, 2 * _DIM), lambda g: (g, 0))],
    out_specs=pl.BlockSpec((_BATCH // 8, _DIM), lambda g: (g, 0)),
)


def kernel(style_id, embed_weight):
    padded = _emb(style_id.astype(jnp.int32), embed_weight)
    return lax.slice(padded, (0, 0), (_BATCH, _DIM))


# speedup vs baseline: 1.3824x; 1.0037x over previous
import functools

import jax
import jax.numpy as jnp
from jax import lax
from jax.experimental import pallas as pl
from jax.experimental.pallas import tpu as pltpu, tpu_sc as plsc

_NUM_STYLES = 1000
_DIM = 64
_BATCH = 16384

_NC = 2
_NS = 16
_NW = _NC * _NS
_BPW = _BATCH // _NW
_CHUNK = 128
_NCHUNK = _BPW // _CHUNK


def _emb_body(idx_hbm, table_hbm, out_hbm, table_s, idx_v, rows_v, gsem, osem):
    cid = lax.axis_index("c")
    sid = lax.axis_index("s")
    base = (sid * _NC + cid) * _BPW

    @pl.when(sid == 0)
    def _stage_table():
        pltpu.sync_copy(table_hbm, table_s)

    pltpu.sync_copy(idx_hbm.at[pl.ds(base, _BPW)], idx_v)
    plsc.subcore_barrier()

    @pl.loop(0, _NCHUNK)
    def _gather(j):
        pltpu.async_copy(
            table_s.at[idx_v.at[pl.ds(j * _CHUNK, _CHUNK)]],
            rows_v.at[pl.ds(j * _CHUNK, _CHUNK)],
            gsem,
        )

    @pl.loop(0, _NCHUNK)
    def _drain(j):
        pltpu.make_async_copy(
            table_s.at[idx_v.at[pl.ds(j * _CHUNK, _CHUNK)]],
            rows_v.at[pl.ds(j * _CHUNK, _CHUNK)],
            gsem,
        ).wait()
        pltpu.async_copy(
            rows_v.at[pl.ds(j * _CHUNK, _CHUNK)],
            out_hbm.at[pl.ds(base + j * _CHUNK, _CHUNK), pl.ds(0, _DIM)],
            osem,
        )

    @pl.loop(0, _NCHUNK)
    def _finish(j):
        pltpu.make_async_copy(
            rows_v.at[pl.ds(j * _CHUNK, _CHUNK)],
            out_hbm.at[pl.ds(base + j * _CHUNK, _CHUNK), pl.ds(0, _DIM)],
            osem,
        ).wait()


_emb = functools.partial(
    pl.kernel,
    out_type=jax.ShapeDtypeStruct((_BATCH, 2 * _DIM), jnp.float32),
    mesh=plsc.VectorSubcoreMesh(core_axis_name="c", subcore_axis_name="s"),
    scratch_types=[
        pltpu.VMEM_SHARED((_NUM_STYLES, _DIM), jnp.float32),
        pltpu.VMEM((_BPW,), jnp.int32),
        pltpu.VMEM((_BPW, _DIM), jnp.float32),
        pltpu.SemaphoreType.DMA,
        pltpu.SemaphoreType.DMA,
    ],
    compiler_params=pltpu.CompilerParams(
        use_tc_tiling_on_sc=False, needs_layout_passes=False
    ),
)(_emb_body)


def _take_left_body(x_ref, o_ref):
    o_ref[...] = x_ref[:, :_DIM]


_take_left = pl.pallas_call(
    _take_left_body,
    out_shape=jax.ShapeDtypeStruct((_BATCH, _DIM), jnp.float32),
    grid=(8,),
    in_specs=[pl.BlockSpec((_BATCH // 8, 2 * _DIM), lambda g: (g, 0))],
    out_specs=pl.BlockSpec((_BATCH // 8, _DIM), lambda g: (g, 0)),
)


def kernel(style_id, embed_weight):
    padded = _emb(style_id.astype(jnp.int32), embed_weight)
    return lax.slice(padded, (0, 0), (_BATCH, _DIM))


# speedup vs baseline: 1.3845x; 1.0015x over previous
import functools

import jax
import jax.numpy as jnp
from jax import lax
from jax.experimental import pallas as pl
from jax.experimental.pallas import tpu as pltpu, tpu_sc as plsc

_NUM_STYLES = 1000
_DIM = 64
_BATCH = 16384

_NC = 2
_NS = 16
_NW = _NC * _NS
_BPW = _BATCH // _NW
_CHUNK = 64
_NCHUNK = _BPW // _CHUNK


def _emb_body(idx_hbm, table_hbm, out_hbm, table_s, idx_v, rows_v, gsem, osem):
    cid = lax.axis_index("c")
    sid = lax.axis_index("s")
    base = (sid * _NC + cid) * _BPW

    @pl.when(sid == 0)
    def _stage_table():
        pltpu.sync_copy(table_hbm, table_s)

    pltpu.sync_copy(idx_hbm.at[pl.ds(base, _BPW)], idx_v)
    plsc.subcore_barrier()

    @pl.loop(0, _NCHUNK)
    def _gather(j):
        pltpu.async_copy(
            table_s.at[idx_v.at[pl.ds(j * _CHUNK, _CHUNK)]],
            rows_v.at[pl.ds(j * _CHUNK, _CHUNK)],
            gsem,
        )

    @pl.loop(0, _NCHUNK)
    def _drain(j):
        pltpu.make_async_copy(
            table_s.at[idx_v.at[pl.ds(j * _CHUNK, _CHUNK)]],
            rows_v.at[pl.ds(j * _CHUNK, _CHUNK)],
            gsem,
        ).wait()
        pltpu.async_copy(
            rows_v.at[pl.ds(j * _CHUNK, _CHUNK)],
            out_hbm.at[pl.ds(base + j * _CHUNK, _CHUNK), pl.ds(0, _DIM)],
            osem,
        )

    @pl.loop(0, _NCHUNK)
    def _finish(j):
        pltpu.make_async_copy(
            rows_v.at[pl.ds(j * _CHUNK, _CHUNK)],
            out_hbm.at[pl.ds(base + j * _CHUNK, _CHUNK), pl.ds(0, _DIM)],
            osem,
        ).wait()


_emb = functools.partial(
    pl.kernel,
    out_type=jax.ShapeDtypeStruct((_BATCH, 2 * _DIM), jnp.float32),
    mesh=plsc.VectorSubcoreMesh(core_axis_name="c", subcore_axis_name="s"),
    scratch_types=[
        pltpu.VMEM_SHARED((_NUM_STYLES, _DIM), jnp.float32),
        pltpu.VMEM((_BPW,), jnp.int32),
        pltpu.VMEM((_BPW, _DIM), jnp.float32),
        pltpu.SemaphoreType.DMA,
        pltpu.SemaphoreType.DMA,
    ],
    compiler_params=pltpu.CompilerParams(
        use_tc_tiling_on_sc=False, needs_layout_passes=False
    ),
)(_emb_body)


def _take_left_body(x_ref, o_ref):
    o_ref[...] = x_ref[:, :_DIM]


_take_left = pl.pallas_call(
    _take_left_body,
    out_shape=jax.ShapeDtypeStruct((_BATCH, _DIM), jnp.float32),
    grid=(8,),
    in_specs=[pl.BlockSpec((_BATCH // 8, 2 * _DIM), lambda g: (g, 0))],
    out_specs=pl.BlockSpec((_BATCH // 8, _DIM), lambda g: (g, 0)),
)


def kernel(style_id, embed_weight):
    padded = _emb(style_id.astype(jnp.int32), embed_weight)
    return lax.slice(padded, (0, 0), (_BATCH, _DIM))
